# Initial kernel scaffold; baseline (speedup 1.0000x reference)
import jax
import jax.numpy as jnp
from jax.experimental import pallas as pl


def kernel(h, top_k_experts, expert_gate, scales):
    raise NotImplementedError("write your pallas kernel here")


# speedup vs baseline: 1.3939x; 1.3939x over previous
import functools

import jax
import jax.numpy as jnp
from jax.experimental import pallas as pl

T = 8192
D = 4096
K = 8
E = 64
BT = 512


def _body(idx_ref, gate_ref, scales_ref, h_ref, out_ref):
    idx = idx_ref[...]
    gate = gate_ref[...]
    sc = scales_ref[...]
    iota_e = jax.lax.broadcasted_iota(jnp.int32, (1, E), 1)
    eff = jnp.zeros((idx.shape[0], 1), jnp.float32)
    for k in range(K):
        cmp = (idx[:, k : k + 1] == iota_e).astype(jnp.float32)
        sk = jnp.sum(cmp * sc, axis=1, keepdims=True)
        eff = eff + gate[:, k : k + 1] * sk
    out_ref[...] = eff * h_ref[...]


@jax.jit
def kernel(h, top_k_experts, expert_gate, scales):
    idx = top_k_experts.astype(jnp.int32)
    sc2 = scales.reshape(1, E)
    grid = (T // BT,)
    return pl.pallas_call(
        _body,
        grid=grid,
        in_specs=[
            pl.BlockSpec((BT, K), lambda i: (i, 0)),
            pl.BlockSpec((BT, K), lambda i: (i, 0)),
            pl.BlockSpec((1, E), lambda i: (0, 0)),
            pl.BlockSpec((BT, D), lambda i: (i, 0)),
        ],
        out_specs=pl.BlockSpec((BT, D), lambda i: (i, 0)),
        out_shape=jax.ShapeDtypeStruct((T, D), jnp.float32),
    )(idx, expert_gate, sc2, h)
